# Initial kernel scaffold; baseline (speedup 1.0000x reference)
import jax
import jax.numpy as jnp
from jax.experimental import pallas as pl


def kernel(x, We1, be1, We2, be2, We3, be3, Wn1, bn1, Wn2, bn2, ang_logit, rad_logit, Wr1, br1, Wr2, br2):
    raise NotImplementedError("write your pallas kernel here")


# speedup vs baseline: 10.5764x; 10.5764x over previous
import math
import numpy as np
import jax
import jax.numpy as jnp
from jax.experimental import pallas as pl
from jax.experimental.pallas import tpu as pltpu

ANG_ = 32
RAD_ = 8
EMB_ = 12
HID_ = 24
NCELL = RAD_ * ANG_
BT = 512


def _roll(v, s):
    n = v.shape[-1]
    s = s % n
    if s == 0:
        return v
    return jnp.concatenate([v[..., n - s:], v[..., :n - s]], axis=-1)


def _fused_kernel(xs_ref, We1T_ref, be1_ref, We2T_ref, be2_ref, We3T_ref,
                  be3_ref, Wn1z_ref, Wn1w_ref, bn1_ref, Wn2_ref, bn2_ref,
                  angl_ref, radl_ref, aexp_ref, rexp_ref, Wr1t_ref, br1_ref,
                  Wr2T_ref, br2_ref, out_ref):
    f32 = jnp.float32
    step = xs_ref[...]

    z1 = jnp.tanh(jnp.dot(step, We1T_ref[...], preferred_element_type=f32)
                  + be1_ref[...])
    z2 = jnp.tanh(jnp.dot(z1, We2T_ref[...], preferred_element_type=f32)
                  + be2_ref[...])
    z = jnp.dot(z2, We3T_ref[...], preferred_element_type=f32) + be3_ref[...]
    log_R = z[:, 0:1]
    phi = z[:, 1:2]

    R_norm = jax.nn.sigmoid(log_R)
    phi_norm = (phi + math.pi) / (2.0 * math.pi)
    R_idx = R_norm * (RAD_ - 1)
    phi_idx = phi_norm * ANG_
    r0 = R_idx.astype(jnp.int32)
    p0 = phi_idx.astype(jnp.int32)
    dr = R_idx - r0.astype(f32)
    dp = phi_idx - p0.astype(f32)

    iota = jax.lax.broadcasted_iota(jnp.int32, (BT, NCELL), 1)
    n_acc = jnp.zeros((BT, NCELL), f32)
    w_acc = jnp.zeros((BT, NCELL), f32)
    for dr_o in (0, 1):
        for dp_o in (0, 1):
            wr = (1.0 - dr) if dr_o == 0 else dr
            wp = (1.0 - dp) if dp_o == 0 else dp
            wt = wr * wp
            r_i = jnp.minimum(r0 + dr_o, RAD_ - 1)
            p_i = jnp.mod(p0 + dp_o, ANG_)
            cell = r_i * ANG_ + p_i
            m = ((iota == cell) & (wt > 0.0)).astype(f32)
            n_acc = n_acc + m
            w_acc = w_acc + wt * m

    def _coefs(logits, expand, shift):
        e = jnp.exp(logits - jnp.max(logits, axis=-1, keepdims=True))
        sm = e / jnp.sum(e, axis=-1, keepdims=True)
        w256 = jnp.dot(sm, expand, preferred_element_type=f32)
        wm = _roll(w256, shift)
        wp = _roll(w256, -shift)
        den = jnp.clip(wm + wp, 1e-6, None)
        return wm / den, wp / den

    CAm, CAp = _coefs(angl_ref[...], aexp_ref[...], RAD_)
    CRm, CRp = _coefs(radl_ref[...], rexp_ref[...], ANG_)

    def _mix(h):
        return (h + CAm * _roll(h, RAD_) + CAp * _roll(h, -RAD_)
                + CRm * _roll(h, ANG_) + CRp * _roll(h, -ANG_))

    nt = _mix(n_acc)
    wtl = _mix(w_acc)

    u = jnp.dot(z, Wn1z_ref[...], preferred_element_type=f32)
    Wn2 = Wn2_ref[...]
    accs = [jnp.zeros((BT, NCELL), f32) for _ in range(EMB_)]
    for j in range(HID_):
        t_j = jnp.tanh(nt * u[:, j:j + 1] + wtl * Wn1w_ref[0, j]
                       + bn1_ref[0, j])
        for e in range(EMB_):
            accs[e] = accs[e] + Wn2[e, j] * t_j

    pre = jnp.zeros((BT, HID_), f32)
    for e in range(EMB_):
        h2_e = accs[e] + bn2_ref[0, e]
        pre = pre + jnp.dot(h2_e, Wr1t_ref[e], preferred_element_type=f32)

    outv = jnp.dot(jnp.tanh(pre + br1_ref[...]), Wr2T_ref[...],
                   preferred_element_type=f32) + br2_ref[...]
    out_ref[...] = outv


def kernel(x, We1, be1, We2, be2, We3, be3, Wn1, bn1, Wn2, bn2,
           ang_logit, rad_logit, Wr1, br1, Wr2, br2):
    B = x.shape[0]
    xs = x[:, -1, :]

    c = np.arange(NCELL)
    aexp = jnp.asarray((c // RAD_ == np.arange(ANG_)[:, None])
                       .astype(np.float32))
    rexp = jnp.asarray((c // ANG_ == np.arange(RAD_)[:, None])
                       .astype(np.float32))

    Wr1t = Wr1.reshape(HID_, NCELL, EMB_).transpose(2, 1, 0)

    args = (
        xs,
        We1.T, be1.reshape(1, -1),
        We2.T, be2.reshape(1, -1),
        We3.T, be3.reshape(1, -1),
        Wn1[:, 0:3].T,
        Wn1[:, 3].reshape(1, -1),
        bn1.reshape(1, -1),
        Wn2, bn2.reshape(1, -1),
        ang_logit.reshape(1, -1), rad_logit.reshape(1, -1),
        aexp, rexp,
        Wr1t, br1.reshape(1, -1),
        Wr2.T, br2.reshape(1, -1),
    )

    def rep(a):
        return pl.BlockSpec(a.shape, lambda i: (0,) * a.ndim)

    in_specs = [pl.BlockSpec((BT, 4), lambda i: (i, 0))]
    in_specs += [rep(a) for a in args[1:]]

    return pl.pallas_call(
        _fused_kernel,
        grid=(B // BT,),
        in_specs=in_specs,
        out_specs=pl.BlockSpec((BT, 4), lambda i: (i, 0)),
        out_shape=jax.ShapeDtypeStruct((B, 4), jnp.float32),
        compiler_params=pltpu.CompilerParams(
            dimension_semantics=("parallel",)),
    )(*args)
